# baseline (speedup 1.0000x reference)
import functools

import jax
import jax.numpy as jnp
from jax import lax
from jax.experimental import pallas as pl
from jax.experimental.pallas import tpu as pltpu
from jax.experimental.pallas import tpu_sc as plsc

_N = 10000
_E = 320000
_H = 128
_B = 64

_NC = 2
_NS = 16
_NW = _NC * _NS
_K = 128
_ITERS = 79
_EPW = _ITERS * _K
_EPAD = _NW * _EPW
_NA = _N + 8
_STRIPE = 624
_STRIPE_LAST = _N - 15 * _STRIPE


@functools.lru_cache(maxsize=None)
def _make_sc_agg():
    mesh = plsc.VectorSubcoreMesh(core_axis_name="c", subcore_axis_name="s")

    @functools.partial(
        pl.kernel,
        mesh=mesh,
        out_type=jax.ShapeDtypeStruct((_NC, _N, _H), jnp.float32),
        scratch_types=[
            pltpu.VMEM((_K,), jnp.int32),
            pltpu.VMEM((_K,), jnp.int32),
            pltpu.VMEM((_K,), jnp.int32),
            pltpu.VMEM((_K,), jnp.int32),
            pltpu.VMEM((_K, _H), jnp.float32),
            pltpu.VMEM_SHARED((_NA, _H), jnp.float32),
            pltpu.SemaphoreType.DMA,
            pltpu.SemaphoreType.DMA,
            pltpu.SemaphoreType.DMA,
        ],
    )
    def agg(h_hbm, src_hbm, dst_hbm, zeros_hbm, out_hbm,
            isa, isb, ida, idb, rows_v, acc_sh, sem_g, sem_ia, sem_ib):
        c = lax.axis_index("c")
        s = lax.axis_index("s")
        wid = s * _NC + c
        r0 = s * _STRIPE

        @pl.when(s < _NS - 1)
        def _():
            pltpu.sync_copy(zeros_hbm.at[pl.ds(0, _STRIPE)],
                            acc_sh.at[pl.ds(r0, _STRIPE)])

        @pl.when(s == _NS - 1)
        def _():
            pltpu.sync_copy(zeros_hbm, acc_sh.at[pl.ds(r0, _STRIPE_LAST)])

        plsc.subcore_barrier()

        pltpu.sync_copy(src_hbm.at[wid, 0], isa)
        pltpu.sync_copy(dst_hbm.at[wid, 0], ida)

        def step(j, cur_is, cur_id, nxt_is, nxt_id, cur_isem, nxt_isem):
            @pl.when(j > 0)
            def _():
                pltpu.make_async_copy(src_hbm.at[0, 0], cur_is, cur_isem).wait()
                pltpu.make_async_copy(dst_hbm.at[0, 0], cur_id, cur_isem).wait()

            pltpu.async_copy(h_hbm.at[cur_is], rows_v, sem_g).wait()

            @pl.when(j + 1 < _ITERS)
            def _():
                pltpu.async_copy(src_hbm.at[wid, j + 1], nxt_is, nxt_isem)
                pltpu.async_copy(dst_hbm.at[wid, j + 1], nxt_id, nxt_isem)

            pltpu.sync_copy(rows_v, acc_sh.at[cur_id], add=True)

        def body(j, carry):
            @pl.when(j % 2 == 0)
            def _():
                step(j, isa, ida, isb, idb, sem_ia, sem_ib)

            @pl.when(j % 2 == 1)
            def _():
                step(j, isb, idb, isa, ida, sem_ib, sem_ia)

            return carry

        lax.fori_loop(0, _ITERS, body, 0)
        plsc.subcore_barrier()

        @pl.when(s < _NS - 1)
        def _():
            pltpu.sync_copy(acc_sh.at[pl.ds(r0, _STRIPE)],
                            out_hbm.at[c, pl.ds(r0, _STRIPE)])

        @pl.when(s == _NS - 1)
        def _():
            pltpu.sync_copy(acc_sh.at[pl.ds(r0, _STRIPE_LAST)],
                            out_hbm.at[c, pl.ds(r0, _STRIPE_LAST)])

    return agg


def _sc_aggregate(h, src, dst, zeros):
    return _make_sc_agg()(h, src, dst, zeros)


_EB = 16000
_NEB = _E // _EB
_Q = 80


def _deg_body(d1_ref, d2_ref, out_ref, acc_s):
    i = pl.program_id(0)

    @pl.when(i == 0)
    def _():
        acc_s[...] = jnp.zeros_like(acc_s)

    q = d1_ref[0] // 128
    r = d2_ref[0] % 128
    oq = (lax.broadcasted_iota(jnp.int32, (_Q, _EB), 0)
          == jnp.broadcast_to(q, (_Q, _EB))).astype(jnp.bfloat16)
    orr = (lax.broadcasted_iota(jnp.int32, (_EB, 128), 1)
           == jnp.broadcast_to(r, (_EB, 128))).astype(jnp.bfloat16)
    acc_s[...] += jnp.dot(oq, orr, preferred_element_type=jnp.float32)

    @pl.when(i == _NEB - 1)
    def _():
        out_ref[...] = acc_s[...]


def _tc_deg(d1, d2):
    return pl.pallas_call(
        _deg_body,
        grid=(_NEB,),
        in_specs=[
            pl.BlockSpec((1, 1, _EB), lambda i: (i, 0, 0)),
            pl.BlockSpec((1, _EB, 1), lambda i: (i, 0, 0)),
        ],
        out_specs=pl.BlockSpec((_Q, 128), lambda i: (0, 0)),
        out_shape=jax.ShapeDtypeStruct((_Q, 128), jnp.float32),
        scratch_shapes=[pltpu.VMEM((_Q, 128), jnp.float32)],
    )(d1, d2)


_R = 1000
_NB = _N // _R


def _norm_factors(st_ref):
    mu = st_ref[0:1, :] * (1.0 / _N)
    var = st_ref[1:2, :] * (1.0 / _N) - mu * mu
    rstd = jax.lax.rsqrt(var + 1e-5)
    return mu, rstd


def _layer_body(p_ref, d_ref, h_ref, wl_ref, bl_ref, wr_ref, y_ref, st_ref,
                st_s):
    i = pl.program_id(0)
    scale = 1.0 / jnp.maximum(d_ref[...], 1.0)
    agg = (p_ref[0] + p_ref[1]) * scale
    y = jnp.dot(agg, wl_ref[...], preferred_element_type=jnp.float32)
    y += jnp.dot(h_ref[...], wr_ref[...], preferred_element_type=jnp.float32)
    y = jnp.maximum(y + bl_ref[...], 0.0)
    y_ref[...] = y

    @pl.when(i == 0)
    def _():
        st_s[...] = jnp.zeros_like(st_s)

    st_s[0:1, :] += jnp.sum(y, axis=0, keepdims=True)
    st_s[1:2, :] += jnp.sum(y * y, axis=0, keepdims=True)

    @pl.when(i == _NB - 1)
    def _():
        st_ref[...] = st_s[...]


def _tc_layer(p, deg2d, h, wl, bl, wr):
    return pl.pallas_call(
        _layer_body,
        grid=(_NB,),
        in_specs=[
            pl.BlockSpec((2, _R, _H), lambda i: (0, i, 0)),
            pl.BlockSpec((_R, 1), lambda i: (i, 0)),
            pl.BlockSpec((_R, _H), lambda i: (i, 0)),
            pl.BlockSpec((_H, _H), lambda i: (0, 0)),
            pl.BlockSpec((1, _H), lambda i: (0, 0)),
            pl.BlockSpec((_H, _H), lambda i: (0, 0)),
        ],
        out_specs=[
            pl.BlockSpec((_R, _H), lambda i: (i, 0)),
            pl.BlockSpec((2, _H), lambda i: (0, 0)),
        ],
        out_shape=[
            jax.ShapeDtypeStruct((_N, _H), jnp.float32),
            jax.ShapeDtypeStruct((2, _H), jnp.float32),
        ],
        scratch_shapes=[pltpu.VMEM((2, _H), jnp.float32)],
    )(p, deg2d, h, wl, bl.reshape(1, _H), wr)


def _norm_body(y_ref, st_ref, g_ref, be_ref, out_ref):
    mu, rstd = _norm_factors(st_ref)
    out_ref[...] = (y_ref[...] - mu) * rstd * g_ref[...] + be_ref[...]


def _tc_norm(y, st, g, be):
    return pl.pallas_call(
        _norm_body,
        grid=(_NB,),
        in_specs=[
            pl.BlockSpec((_R, _H), lambda i: (i, 0)),
            pl.BlockSpec((2, _H), lambda i: (0, 0)),
            pl.BlockSpec((1, _H), lambda i: (0, 0)),
            pl.BlockSpec((1, _H), lambda i: (0, 0)),
        ],
        out_specs=pl.BlockSpec((_R, _H), lambda i: (i, 0)),
        out_shape=jax.ShapeDtypeStruct((_N, _H), jnp.float32),
    )(y, st, g.reshape(1, _H), be.reshape(1, _H))


def _pool_body(y_ref, st_ref, g_ref, be_ref, b_ref, max_ref, mean_ref,
               maxs_s, sums_s, cnt_s):
    i = pl.program_id(0)

    @pl.when(i == 0)
    def _():
        maxs_s[...] = jnp.full_like(maxs_s, -jnp.inf)
        sums_s[...] = jnp.zeros_like(sums_s)
        cnt_s[...] = jnp.zeros_like(cnt_s)

    mu, rstd = _norm_factors(st_ref)
    hv = (y_ref[...] - mu) * rstd * g_ref[...] + be_ref[...]
    bv = b_ref[...]
    bmin = b_ref[0, 0]
    bmax = b_ref[_R - 1, 0]

    def body(b, carry):
        mask = bv == b
        mx = jnp.max(jnp.where(mask, hv, -jnp.inf), axis=0, keepdims=True)
        sm = jnp.sum(jnp.where(mask, hv, 0.0), axis=0, keepdims=True)
        ct = jnp.sum(mask.astype(jnp.float32))
        maxs_s[pl.ds(b, 1), :] = jnp.maximum(maxs_s[pl.ds(b, 1), :], mx)
        sums_s[pl.ds(b, 1), :] += sm
        cnt_s[pl.ds(b, 1), :] += ct
        return carry

    lax.fori_loop(bmin, bmax + 1, body, 0)

    @pl.when(i == _NB - 1)
    def _():
        max_ref[...] = maxs_s[...]
        mean_ref[...] = sums_s[...] / jnp.maximum(cnt_s[...], 1.0)


def _tc_pool(y, st, g, be, batch2d):
    return pl.pallas_call(
        _pool_body,
        grid=(_NB,),
        in_specs=[
            pl.BlockSpec((_R, _H), lambda i: (i, 0)),
            pl.BlockSpec((2, _H), lambda i: (0, 0)),
            pl.BlockSpec((1, _H), lambda i: (0, 0)),
            pl.BlockSpec((1, _H), lambda i: (0, 0)),
            pl.BlockSpec((_R, 1), lambda i: (i, 0)),
        ],
        out_specs=[
            pl.BlockSpec((_B, _H), lambda i: (0, 0)),
            pl.BlockSpec((_B, _H), lambda i: (0, 0)),
        ],
        out_shape=[
            jax.ShapeDtypeStruct((_B, _H), jnp.float32),
            jax.ShapeDtypeStruct((_B, _H), jnp.float32),
        ],
        scratch_shapes=[
            pltpu.VMEM((_B, _H), jnp.float32),
            pltpu.VMEM((_B, _H), jnp.float32),
            pltpu.VMEM((_B, _H), jnp.float32),
        ],
    )(y, st, g.reshape(1, _H), be.reshape(1, _H), batch2d)


def _head_body(first_ref, xmax_ref, xmean_ref, y_ref, st_ref, g_ref, be_ref,
               w1_ref, b1_ref, w2_ref, b2_ref, out_ref, x3_s):
    def gather(i, carry):
        r = first_ref[i]
        x3_s[pl.ds(i, 1), :] = y_ref[pl.ds(r, 1), :]
        return carry

    lax.fori_loop(0, _B, gather, 0)
    mu, rstd = _norm_factors(st_ref)
    x3 = (x3_s[...] - mu) * rstd * g_ref[...] + be_ref[...]
    z = jnp.concatenate([xmax_ref[...], xmean_ref[...], x3], axis=1)
    z = jnp.dot(z, w1_ref[...], preferred_element_type=jnp.float32) + b1_ref[...]
    z = jnp.dot(z, w2_ref[...], preferred_element_type=jnp.float32) + b2_ref[...]
    m = jnp.max(z, axis=1, keepdims=True)
    lse = jnp.log(jnp.sum(jnp.exp(z - m), axis=1, keepdims=True)) + m
    out_ref[...] = z - lse


def _tc_head(first, xmax, xmean, y, st, g, be, w1, b1, w2, b2):
    nout = b2.shape[0]
    return pl.pallas_call(
        _head_body,
        in_specs=[
            pl.BlockSpec(memory_space=pltpu.SMEM),
            pl.BlockSpec((_B, _H), lambda: (0, 0)),
            pl.BlockSpec((_B, _H), lambda: (0, 0)),
            pl.BlockSpec((_N, _H), lambda: (0, 0)),
            pl.BlockSpec((2, _H), lambda: (0, 0)),
            pl.BlockSpec((1, _H), lambda: (0, 0)),
            pl.BlockSpec((1, _H), lambda: (0, 0)),
            pl.BlockSpec((3 * _H, 3 * _H), lambda: (0, 0)),
            pl.BlockSpec((1, 3 * _H), lambda: (0, 0)),
            pl.BlockSpec((3 * _H, nout), lambda: (0, 0)),
            pl.BlockSpec((1, nout), lambda: (0, 0)),
        ],
        out_specs=pl.BlockSpec((_B, nout), lambda: (0, 0)),
        out_shape=jax.ShapeDtypeStruct((_B, nout), jnp.float32),
        scratch_shapes=[pltpu.VMEM((_B, _H), jnp.float32)],
    )(first, xmax, xmean, y, st, g.reshape(1, _H), be.reshape(1, _H),
      w1, b1.reshape(1, -1), w2, b2.reshape(1, -1))


def kernel(x, edge_index, batch, Wl0, bl0, Wr0, g0, be0, Wl1, bl1, Wr1, g1,
           be1, W1, b1, W2, b2):
    srcf = edge_index[0].astype(jnp.int32)
    dstf = edge_index[1].astype(jnp.int32)
    npad = _EPAD - _E
    src = jnp.concatenate(
        [srcf, jnp.zeros((npad,), jnp.int32)]).reshape(_NW, _ITERS, _K)
    dst = jnp.concatenate(
        [dstf, jnp.full((npad,), _N, jnp.int32)]).reshape(_NW, _ITERS, _K)
    zeros128 = jnp.zeros((_STRIPE_LAST, _H), jnp.float32)

    degqr = _tc_deg(dstf.reshape(_NEB, 1, _EB), dstf.reshape(_NEB, _EB, 1))
    deg2d = degqr.reshape(_Q * 128, 1)[:_N]

    p0 = _sc_aggregate(x, src, dst, zeros128)
    y0, st0 = _tc_layer(p0, deg2d, x, Wl0, bl0, Wr0)
    h1 = _tc_norm(y0, st0, g0, be0)

    p1 = _sc_aggregate(h1, src, dst, zeros128)
    y1, st1 = _tc_layer(p1, deg2d, h1, Wl1, bl1, Wr1)

    batch2d = batch.astype(jnp.int32).reshape(_N, 1)
    xmax, xmean = _tc_pool(y1, st1, g1, be1, batch2d)
    first = jnp.searchsorted(batch.astype(jnp.int32),
                             jnp.arange(_B, dtype=jnp.int32)).astype(jnp.int32)
    return _tc_head(first, xmax, xmean, y1, st1, g1, be1, W1, b1, W2, b2)


# speedup vs baseline: 1.2801x; 1.2801x over previous
import functools

import jax
import jax.numpy as jnp
from jax import lax
from jax.experimental import pallas as pl
from jax.experimental.pallas import tpu as pltpu
from jax.experimental.pallas import tpu_sc as plsc

_N = 10000
_E = 320000
_H = 128
_B = 64

_NC = 2
_NS = 16
_NW = _NC * _NS
_K = 80
_EPW = _E // _NW
_ITERS = _EPW // _K
_NA = _N
_STRIPE = 624
_STRIPE_LAST = _N - 15 * _STRIPE


@functools.lru_cache(maxsize=None)
def _make_sc_agg():
    mesh = plsc.VectorSubcoreMesh(core_axis_name="c", subcore_axis_name="s")

    @functools.partial(
        pl.kernel,
        mesh=mesh,
        out_type=jax.ShapeDtypeStruct((_NC, _N, _H), jnp.float32),
        scratch_types=[
            pltpu.VMEM((_ITERS, _K), jnp.int32),
            pltpu.VMEM((_ITERS, _K), jnp.int32),
            pltpu.VMEM((_K, _H), jnp.float32),
            pltpu.VMEM_SHARED((_NA, _H), jnp.float32),
            pltpu.SemaphoreType.DMA,
        ],
    )
    def agg(h_hbm, src_hbm, dst_hbm, zeros_hbm, out_hbm,
            src_v, dst_v, rows_v, acc_sh, sem_g):
        c = lax.axis_index("c")
        s = lax.axis_index("s")
        wid = s * _NC + c
        r0 = s * _STRIPE

        @pl.when(s < _NS - 1)
        def _():
            pltpu.sync_copy(zeros_hbm.at[pl.ds(0, _STRIPE)],
                            acc_sh.at[pl.ds(r0, _STRIPE)])

        @pl.when(s == _NS - 1)
        def _():
            pltpu.sync_copy(zeros_hbm, acc_sh.at[pl.ds(r0, _STRIPE_LAST)])

        pltpu.sync_copy(src_hbm.at[wid], src_v)
        pltpu.sync_copy(dst_hbm.at[wid], dst_v)
        plsc.subcore_barrier()

        def body(j, carry):
            pltpu.async_copy(h_hbm.at[src_v.at[j]], rows_v, sem_g).wait()
            pltpu.sync_copy(rows_v, acc_sh.at[dst_v.at[j]], add=True)
            return carry

        lax.fori_loop(0, _ITERS, body, 0)
        plsc.subcore_barrier()

        @pl.when(s < _NS - 1)
        def _():
            pltpu.sync_copy(acc_sh.at[pl.ds(r0, _STRIPE)],
                            out_hbm.at[c, pl.ds(r0, _STRIPE)])

        @pl.when(s == _NS - 1)
        def _():
            pltpu.sync_copy(acc_sh.at[pl.ds(r0, _STRIPE_LAST)],
                            out_hbm.at[c, pl.ds(r0, _STRIPE_LAST)])

    return agg


def _sc_aggregate(h, src, dst, zeros):
    return _make_sc_agg()(h, src, dst, zeros)


_EB = 16000
_NEB = _E // _EB
_Q = 80


def _deg_body(d1_ref, d2_ref, out_ref, acc_s):
    i = pl.program_id(0)

    @pl.when(i == 0)
    def _():
        acc_s[...] = jnp.zeros_like(acc_s)

    q = d1_ref[0] // 128
    r = d2_ref[0] % 128
    oq = (lax.broadcasted_iota(jnp.int32, (_Q, _EB), 0)
          == jnp.broadcast_to(q, (_Q, _EB))).astype(jnp.bfloat16)
    orr = (lax.broadcasted_iota(jnp.int32, (_EB, 128), 1)
           == jnp.broadcast_to(r, (_EB, 128))).astype(jnp.bfloat16)
    acc_s[...] += jnp.dot(oq, orr, preferred_element_type=jnp.float32)

    @pl.when(i == _NEB - 1)
    def _():
        out_ref[...] = acc_s[...]


def _tc_deg(d1, d2):
    return pl.pallas_call(
        _deg_body,
        grid=(_NEB,),
        in_specs=[
            pl.BlockSpec((1, 1, _EB), lambda i: (i, 0, 0)),
            pl.BlockSpec((1, _EB, 1), lambda i: (i, 0, 0)),
        ],
        out_specs=pl.BlockSpec((_Q, 128), lambda i: (0, 0)),
        out_shape=jax.ShapeDtypeStruct((_Q, 128), jnp.float32),
        scratch_shapes=[pltpu.VMEM((_Q, 128), jnp.float32)],
    )(d1, d2)


_R = 1000
_NB = _N // _R


def _norm_factors(st_ref):
    mu = st_ref[0:1, :] * (1.0 / _N)
    var = st_ref[1:2, :] * (1.0 / _N) - mu * mu
    rstd = jax.lax.rsqrt(var + 1e-5)
    return mu, rstd


def _layer_body(p_ref, d_ref, h_ref, wl_ref, bl_ref, wr_ref, y_ref, st_ref,
                st_s):
    i = pl.program_id(0)
    scale = 1.0 / jnp.maximum(d_ref[...], 1.0)
    agg = (p_ref[0] + p_ref[1]) * scale
    y = jnp.dot(agg, wl_ref[...], preferred_element_type=jnp.float32)
    y += jnp.dot(h_ref[...], wr_ref[...], preferred_element_type=jnp.float32)
    y = jnp.maximum(y + bl_ref[...], 0.0)
    y_ref[...] = y

    @pl.when(i == 0)
    def _():
        st_s[...] = jnp.zeros_like(st_s)

    st_s[0:1, :] += jnp.sum(y, axis=0, keepdims=True)
    st_s[1:2, :] += jnp.sum(y * y, axis=0, keepdims=True)

    @pl.when(i == _NB - 1)
    def _():
        st_ref[...] = st_s[...]


def _tc_layer(p, deg2d, h, wl, bl, wr):
    return pl.pallas_call(
        _layer_body,
        grid=(_NB,),
        in_specs=[
            pl.BlockSpec((2, _R, _H), lambda i: (0, i, 0)),
            pl.BlockSpec((_R, 1), lambda i: (i, 0)),
            pl.BlockSpec((_R, _H), lambda i: (i, 0)),
            pl.BlockSpec((_H, _H), lambda i: (0, 0)),
            pl.BlockSpec((1, _H), lambda i: (0, 0)),
            pl.BlockSpec((_H, _H), lambda i: (0, 0)),
        ],
        out_specs=[
            pl.BlockSpec((_R, _H), lambda i: (i, 0)),
            pl.BlockSpec((2, _H), lambda i: (0, 0)),
        ],
        out_shape=[
            jax.ShapeDtypeStruct((_N, _H), jnp.float32),
            jax.ShapeDtypeStruct((2, _H), jnp.float32),
        ],
        scratch_shapes=[pltpu.VMEM((2, _H), jnp.float32)],
    )(p, deg2d, h, wl, bl.reshape(1, _H), wr)


def _norm_body(y_ref, st_ref, g_ref, be_ref, out_ref):
    mu, rstd = _norm_factors(st_ref)
    out_ref[...] = (y_ref[...] - mu) * rstd * g_ref[...] + be_ref[...]


def _tc_norm(y, st, g, be):
    return pl.pallas_call(
        _norm_body,
        grid=(_NB,),
        in_specs=[
            pl.BlockSpec((_R, _H), lambda i: (i, 0)),
            pl.BlockSpec((2, _H), lambda i: (0, 0)),
            pl.BlockSpec((1, _H), lambda i: (0, 0)),
            pl.BlockSpec((1, _H), lambda i: (0, 0)),
        ],
        out_specs=pl.BlockSpec((_R, _H), lambda i: (i, 0)),
        out_shape=jax.ShapeDtypeStruct((_N, _H), jnp.float32),
    )(y, st, g.reshape(1, _H), be.reshape(1, _H))


def _pool_body(y_ref, st_ref, g_ref, be_ref, b_ref, max_ref, mean_ref,
               maxs_s, sums_s, cnt_s):
    i = pl.program_id(0)

    @pl.when(i == 0)
    def _():
        maxs_s[...] = jnp.full_like(maxs_s, -jnp.inf)
        sums_s[...] = jnp.zeros_like(sums_s)
        cnt_s[...] = jnp.zeros_like(cnt_s)

    mu, rstd = _norm_factors(st_ref)
    hv = (y_ref[...] - mu) * rstd * g_ref[...] + be_ref[...]
    bv = b_ref[...]
    bmin = b_ref[0, 0]
    bmax = b_ref[_R - 1, 0]

    def body(b, carry):
        mask = bv == b
        mx = jnp.max(jnp.where(mask, hv, -jnp.inf), axis=0, keepdims=True)
        sm = jnp.sum(jnp.where(mask, hv, 0.0), axis=0, keepdims=True)
        ct = jnp.sum(mask.astype(jnp.float32))
        maxs_s[pl.ds(b, 1), :] = jnp.maximum(maxs_s[pl.ds(b, 1), :], mx)
        sums_s[pl.ds(b, 1), :] += sm
        cnt_s[pl.ds(b, 1), :] += ct
        return carry

    lax.fori_loop(bmin, bmax + 1, body, 0)

    @pl.when(i == _NB - 1)
    def _():
        max_ref[...] = maxs_s[...]
        mean_ref[...] = sums_s[...] / jnp.maximum(cnt_s[...], 1.0)


def _tc_pool(y, st, g, be, batch2d):
    return pl.pallas_call(
        _pool_body,
        grid=(_NB,),
        in_specs=[
            pl.BlockSpec((_R, _H), lambda i: (i, 0)),
            pl.BlockSpec((2, _H), lambda i: (0, 0)),
            pl.BlockSpec((1, _H), lambda i: (0, 0)),
            pl.BlockSpec((1, _H), lambda i: (0, 0)),
            pl.BlockSpec((_R, 1), lambda i: (i, 0)),
        ],
        out_specs=[
            pl.BlockSpec((_B, _H), lambda i: (0, 0)),
            pl.BlockSpec((_B, _H), lambda i: (0, 0)),
        ],
        out_shape=[
            jax.ShapeDtypeStruct((_B, _H), jnp.float32),
            jax.ShapeDtypeStruct((_B, _H), jnp.float32),
        ],
        scratch_shapes=[
            pltpu.VMEM((_B, _H), jnp.float32),
            pltpu.VMEM((_B, _H), jnp.float32),
            pltpu.VMEM((_B, _H), jnp.float32),
        ],
    )(y, st, g.reshape(1, _H), be.reshape(1, _H), batch2d)


def _head_body(first_ref, xmax_ref, xmean_ref, y_ref, st_ref, g_ref, be_ref,
               w1_ref, b1_ref, w2_ref, b2_ref, out_ref, x3_s):
    def gather(i, carry):
        r = first_ref[i]
        x3_s[pl.ds(i, 1), :] = y_ref[pl.ds(r, 1), :]
        return carry

    lax.fori_loop(0, _B, gather, 0)
    mu, rstd = _norm_factors(st_ref)
    x3 = (x3_s[...] - mu) * rstd * g_ref[...] + be_ref[...]
    z = jnp.concatenate([xmax_ref[...], xmean_ref[...], x3], axis=1)
    z = jnp.dot(z, w1_ref[...], preferred_element_type=jnp.float32) + b1_ref[...]
    z = jnp.dot(z, w2_ref[...], preferred_element_type=jnp.float32) + b2_ref[...]
    m = jnp.max(z, axis=1, keepdims=True)
    lse = jnp.log(jnp.sum(jnp.exp(z - m), axis=1, keepdims=True)) + m
    out_ref[...] = z - lse


def _tc_head(first, xmax, xmean, y, st, g, be, w1, b1, w2, b2):
    nout = b2.shape[0]
    return pl.pallas_call(
        _head_body,
        in_specs=[
            pl.BlockSpec(memory_space=pltpu.SMEM),
            pl.BlockSpec((_B, _H), lambda: (0, 0)),
            pl.BlockSpec((_B, _H), lambda: (0, 0)),
            pl.BlockSpec((_N, _H), lambda: (0, 0)),
            pl.BlockSpec((2, _H), lambda: (0, 0)),
            pl.BlockSpec((1, _H), lambda: (0, 0)),
            pl.BlockSpec((1, _H), lambda: (0, 0)),
            pl.BlockSpec((3 * _H, 3 * _H), lambda: (0, 0)),
            pl.BlockSpec((1, 3 * _H), lambda: (0, 0)),
            pl.BlockSpec((3 * _H, nout), lambda: (0, 0)),
            pl.BlockSpec((1, nout), lambda: (0, 0)),
        ],
        out_specs=pl.BlockSpec((_B, nout), lambda: (0, 0)),
        out_shape=jax.ShapeDtypeStruct((_B, nout), jnp.float32),
        scratch_shapes=[pltpu.VMEM((_B, _H), jnp.float32)],
    )(first, xmax, xmean, y, st, g.reshape(1, _H), be.reshape(1, _H),
      w1, b1.reshape(1, -1), w2, b2.reshape(1, -1))


def kernel(x, edge_index, batch, Wl0, bl0, Wr0, g0, be0, Wl1, bl1, Wr1, g1,
           be1, W1, b1, W2, b2):
    srcf = edge_index[0].astype(jnp.int32)
    dstf = edge_index[1].astype(jnp.int32)
    src = srcf.reshape(_NW, _ITERS, _K)
    dst = dstf.reshape(_NW, _ITERS, _K)
    zeros128 = jnp.zeros((_STRIPE_LAST, _H), jnp.float32)

    degqr = _tc_deg(dstf.reshape(_NEB, 1, _EB), dstf.reshape(_NEB, _EB, 1))
    deg2d = degqr.reshape(_Q * 128, 1)[:_N]

    p0 = _sc_aggregate(x, src, dst, zeros128)
    y0, st0 = _tc_layer(p0, deg2d, x, Wl0, bl0, Wr0)
    h1 = _tc_norm(y0, st0, g0, be0)

    p1 = _sc_aggregate(h1, src, dst, zeros128)
    y1, st1 = _tc_layer(p1, deg2d, h1, Wl1, bl1, Wr1)

    batch2d = batch.astype(jnp.int32).reshape(_N, 1)
    xmax, xmean = _tc_pool(y1, st1, g1, be1, batch2d)
    first = jnp.searchsorted(batch.astype(jnp.int32),
                             jnp.arange(_B, dtype=jnp.int32)).astype(jnp.int32)
    return _tc_head(first, xmax, xmean, y1, st1, g1, be1, W1, b1, W2, b2)


# speedup vs baseline: 1.2812x; 1.0009x over previous
import functools

import jax
import jax.numpy as jnp
from jax import lax
from jax.experimental import pallas as pl
from jax.experimental.pallas import tpu as pltpu
from jax.experimental.pallas import tpu_sc as plsc

_N = 10000
_E = 320000
_H = 128
_B = 64

_NC = 2
_NS = 16
_NW = _NC * _NS
_K = 80
_EPW = _E // _NW
_ITERS = _EPW // _K
_NA = _N
_STRIPE = 624
_STRIPE_LAST = _N - 15 * _STRIPE


@functools.lru_cache(maxsize=None)
def _make_sc_agg():
    mesh = plsc.VectorSubcoreMesh(core_axis_name="c", subcore_axis_name="s")

    @functools.partial(
        pl.kernel,
        mesh=mesh,
        out_type=jax.ShapeDtypeStruct((_NC, _N, _H), jnp.float32),
        scratch_types=[
            pltpu.VMEM((_ITERS, _K), jnp.int32),
            pltpu.VMEM((_ITERS, _K), jnp.int32),
            pltpu.VMEM((_K, _H), jnp.float32),
            pltpu.VMEM_SHARED((_NA, _H), jnp.float32),
            pltpu.SemaphoreType.DMA,
        ],
    )
    def agg(h_hbm, src_hbm, dst_hbm, zeros_hbm, out_hbm,
            src_v, dst_v, rows_v, acc_sh, sem_g):
        c = lax.axis_index("c")
        s = lax.axis_index("s")
        wid = s * _NC + c
        r0 = s * _STRIPE

        @pl.when(s < _NS - 1)
        def _():
            pltpu.sync_copy(zeros_hbm.at[pl.ds(0, _STRIPE)],
                            acc_sh.at[pl.ds(r0, _STRIPE)])

        @pl.when(s == _NS - 1)
        def _():
            pltpu.sync_copy(zeros_hbm, acc_sh.at[pl.ds(r0, _STRIPE_LAST)])

        pltpu.sync_copy(src_hbm.at[wid], src_v)
        pltpu.sync_copy(dst_hbm.at[wid], dst_v)
        plsc.subcore_barrier()

        def body(j, carry):
            pltpu.async_copy(h_hbm.at[src_v.at[j]], rows_v, sem_g).wait()
            pltpu.sync_copy(rows_v, acc_sh.at[dst_v.at[j]], add=True)
            return carry

        lax.fori_loop(0, _ITERS, body, 0)
        plsc.subcore_barrier()

        @pl.when(s < _NS - 1)
        def _():
            pltpu.sync_copy(acc_sh.at[pl.ds(r0, _STRIPE)],
                            out_hbm.at[c, pl.ds(r0, _STRIPE)])

        @pl.when(s == _NS - 1)
        def _():
            pltpu.sync_copy(acc_sh.at[pl.ds(r0, _STRIPE_LAST)],
                            out_hbm.at[c, pl.ds(r0, _STRIPE_LAST)])

    return agg


def _sc_aggregate(h, src, dst, zeros):
    return _make_sc_agg()(h, src, dst, zeros)


_EB = 16000
_NEB = _E // _EB
_Q = 80


def _deg_body(d1_ref, d2_ref, out_ref, acc_s):
    i = pl.program_id(0)

    @pl.when(i == 0)
    def _():
        acc_s[...] = jnp.zeros_like(acc_s)

    q = d1_ref[0] // 128
    r = d2_ref[0] % 128
    oq = (lax.broadcasted_iota(jnp.int32, (_Q, _EB), 0)
          == jnp.broadcast_to(q, (_Q, _EB))).astype(jnp.bfloat16)
    orr = (lax.broadcasted_iota(jnp.int32, (_EB, 128), 1)
           == jnp.broadcast_to(r, (_EB, 128))).astype(jnp.bfloat16)
    acc_s[...] += jnp.dot(oq, orr, preferred_element_type=jnp.float32)

    @pl.when(i == _NEB - 1)
    def _():
        out_ref[...] = acc_s[...]


def _tc_deg(d1, d2):
    return pl.pallas_call(
        _deg_body,
        grid=(_NEB,),
        in_specs=[
            pl.BlockSpec((1, 1, _EB), lambda i: (i, 0, 0)),
            pl.BlockSpec((1, _EB, 1), lambda i: (i, 0, 0)),
        ],
        out_specs=pl.BlockSpec((_Q, 128), lambda i: (0, 0)),
        out_shape=jax.ShapeDtypeStruct((_Q, 128), jnp.float32),
        scratch_shapes=[pltpu.VMEM((_Q, 128), jnp.float32)],
    )(d1, d2)


_R = 1000
_NB = _N // _R


def _norm_factors(st_ref):
    mu = st_ref[0:1, :] * (1.0 / _N)
    var = st_ref[1:2, :] * (1.0 / _N) - mu * mu
    rstd = jax.lax.rsqrt(var + 1e-5)
    return mu, rstd


def _layer_body(p_ref, d_ref, h_ref, wl_ref, bl_ref, wr_ref, y_ref, st_ref,
                st_s):
    i = pl.program_id(0)
    scale = 1.0 / jnp.maximum(d_ref[...], 1.0)
    agg = (p_ref[0] + p_ref[1]) * scale
    y = jnp.dot(agg, wl_ref[...], preferred_element_type=jnp.float32)
    y += jnp.dot(h_ref[...], wr_ref[...], preferred_element_type=jnp.float32)
    y = jnp.maximum(y + bl_ref[...], 0.0)
    y_ref[...] = y

    @pl.when(i == 0)
    def _():
        st_s[...] = jnp.zeros_like(st_s)

    st_s[0:1, :] += jnp.sum(y, axis=0, keepdims=True)
    st_s[1:2, :] += jnp.sum(y * y, axis=0, keepdims=True)

    @pl.when(i == _NB - 1)
    def _():
        st_ref[...] = st_s[...]


def _tc_layer(p, deg2d, h, wl, bl, wr):
    return pl.pallas_call(
        _layer_body,
        grid=(_NB,),
        in_specs=[
            pl.BlockSpec((2, _R, _H), lambda i: (0, i, 0)),
            pl.BlockSpec((_R, 1), lambda i: (i, 0)),
            pl.BlockSpec((_R, _H), lambda i: (i, 0)),
            pl.BlockSpec((_H, _H), lambda i: (0, 0)),
            pl.BlockSpec((1, _H), lambda i: (0, 0)),
            pl.BlockSpec((_H, _H), lambda i: (0, 0)),
        ],
        out_specs=[
            pl.BlockSpec((_R, _H), lambda i: (i, 0)),
            pl.BlockSpec((2, _H), lambda i: (0, 0)),
        ],
        out_shape=[
            jax.ShapeDtypeStruct((_N, _H), jnp.float32),
            jax.ShapeDtypeStruct((2, _H), jnp.float32),
        ],
        scratch_shapes=[pltpu.VMEM((2, _H), jnp.float32)],
    )(p, deg2d, h, wl, bl.reshape(1, _H), wr)


def _norm_body(y_ref, st_ref, g_ref, be_ref, out_ref):
    mu, rstd = _norm_factors(st_ref)
    out_ref[...] = (y_ref[...] - mu) * rstd * g_ref[...] + be_ref[...]


def _tc_norm(y, st, g, be):
    return pl.pallas_call(
        _norm_body,
        grid=(_NB,),
        in_specs=[
            pl.BlockSpec((_R, _H), lambda i: (i, 0)),
            pl.BlockSpec((2, _H), lambda i: (0, 0)),
            pl.BlockSpec((1, _H), lambda i: (0, 0)),
            pl.BlockSpec((1, _H), lambda i: (0, 0)),
        ],
        out_specs=pl.BlockSpec((_R, _H), lambda i: (i, 0)),
        out_shape=jax.ShapeDtypeStruct((_N, _H), jnp.float32),
    )(y, st, g.reshape(1, _H), be.reshape(1, _H))


def _poolhead_body(first_ref, batch_s, y_ref, st_ref, g_ref, be_ref, b2d_ref,
                   w1_ref, b1_ref, w2_ref, b2_ref, out_ref,
                   maxs_s, sums_s, cnt_s, x3_s):
    maxs_s[...] = jnp.full_like(maxs_s, -jnp.inf)
    sums_s[...] = jnp.zeros_like(sums_s)
    cnt_s[...] = jnp.zeros_like(cnt_s)
    mu, rstd = _norm_factors(st_ref)
    gv = g_ref[...]
    bev = be_ref[...]

    def blk(i, carry):
        hv = (y_ref[pl.ds(i * _R, _R), :] - mu) * rstd * gv + bev
        bv = b2d_ref[pl.ds(i * _R, _R), :]
        bmin = batch_s[i * _R]
        bmax = batch_s[i * _R + _R - 1]

        def seg(b, carry2):
            mask = bv == b
            mx = jnp.max(jnp.where(mask, hv, -jnp.inf), axis=0, keepdims=True)
            sm = jnp.sum(jnp.where(mask, hv, 0.0), axis=0, keepdims=True)
            ct = jnp.sum(mask.astype(jnp.float32))
            maxs_s[pl.ds(b, 1), :] = jnp.maximum(maxs_s[pl.ds(b, 1), :], mx)
            sums_s[pl.ds(b, 1), :] += sm
            cnt_s[pl.ds(b, 1), :] += ct
            return carry2

        lax.fori_loop(bmin, bmax + 1, seg, 0)
        return carry

    lax.fori_loop(0, _NB, blk, 0)

    def gather(i, carry):
        r = first_ref[i]
        x3_s[pl.ds(i, 1), :] = y_ref[pl.ds(r, 1), :]
        return carry

    lax.fori_loop(0, _B, gather, 0)
    x3 = (x3_s[...] - mu) * rstd * gv + bev
    xmean = sums_s[...] / jnp.maximum(cnt_s[...], 1.0)
    z = jnp.concatenate([maxs_s[...], xmean, x3], axis=1)
    z = jnp.dot(z, w1_ref[...], preferred_element_type=jnp.float32) + b1_ref[...]
    z = jnp.dot(z, w2_ref[...], preferred_element_type=jnp.float32) + b2_ref[...]
    m = jnp.max(z, axis=1, keepdims=True)
    lse = jnp.log(jnp.sum(jnp.exp(z - m), axis=1, keepdims=True)) + m
    out_ref[...] = z - lse


def _tc_poolhead(first, batch1d, y, st, g, be, batch2d, w1, b1, w2, b2):
    nout = b2.shape[0]
    return pl.pallas_call(
        _poolhead_body,
        in_specs=[
            pl.BlockSpec(memory_space=pltpu.SMEM),
            pl.BlockSpec(memory_space=pltpu.SMEM),
            pl.BlockSpec((_N, _H), lambda: (0, 0)),
            pl.BlockSpec((2, _H), lambda: (0, 0)),
            pl.BlockSpec((1, _H), lambda: (0, 0)),
            pl.BlockSpec((1, _H), lambda: (0, 0)),
            pl.BlockSpec((_N, 1), lambda: (0, 0)),
            pl.BlockSpec((3 * _H, 3 * _H), lambda: (0, 0)),
            pl.BlockSpec((1, 3 * _H), lambda: (0, 0)),
            pl.BlockSpec((3 * _H, nout), lambda: (0, 0)),
            pl.BlockSpec((1, nout), lambda: (0, 0)),
        ],
        out_specs=pl.BlockSpec((_B, nout), lambda: (0, 0)),
        out_shape=jax.ShapeDtypeStruct((_B, nout), jnp.float32),
        scratch_shapes=[
            pltpu.VMEM((_B, _H), jnp.float32),
            pltpu.VMEM((_B, _H), jnp.float32),
            pltpu.VMEM((_B, _H), jnp.float32),
            pltpu.VMEM((_B, _H), jnp.float32),
        ],
    )(first, batch1d, y, st, g.reshape(1, _H), be.reshape(1, _H), batch2d,
      w1, b1.reshape(1, -1), w2, b2.reshape(1, -1))


def kernel(x, edge_index, batch, Wl0, bl0, Wr0, g0, be0, Wl1, bl1, Wr1, g1,
           be1, W1, b1, W2, b2):
    srcf = edge_index[0].astype(jnp.int32)
    dstf = edge_index[1].astype(jnp.int32)
    src = srcf.reshape(_NW, _ITERS, _K)
    dst = dstf.reshape(_NW, _ITERS, _K)
    zeros128 = jnp.zeros((_STRIPE_LAST, _H), jnp.float32)

    degqr = _tc_deg(dstf.reshape(_NEB, 1, _EB), dstf.reshape(_NEB, _EB, 1))
    deg2d = degqr.reshape(_Q * 128, 1)[:_N]

    p0 = _sc_aggregate(x, src, dst, zeros128)
    y0, st0 = _tc_layer(p0, deg2d, x, Wl0, bl0, Wr0)
    h1 = _tc_norm(y0, st0, g0, be0)

    p1 = _sc_aggregate(h1, src, dst, zeros128)
    y1, st1 = _tc_layer(p1, deg2d, h1, Wl1, bl1, Wr1)

    batch1d = batch.astype(jnp.int32)
    batch2d = batch1d.reshape(_N, 1)
    first = jnp.searchsorted(batch1d,
                             jnp.arange(_B, dtype=jnp.int32)).astype(jnp.int32)
    return _tc_poolhead(first, batch1d, y1, st1, g1, be1, batch2d,
                        W1, b1, W2, b2)
